# baseline (speedup 1.0000x reference)
import functools
import math

import jax
import jax.numpy as jnp
from jax import lax
from jax.experimental import pallas as pl
from jax.experimental.pallas import tpu as pltpu
from jax.experimental.pallas import tpu_sc as plsc

N = 2048
D = 1024
F = 4096
E = 8
CAP = 320
ECAP = E * CAP
FBLK = 2048
NFB = F // FBLK
TBLK = 128
NTB = N // TBLK
SBLK = 256
NSB = ECAP // SBLK


def _route_body(x_ref, rw_ref, rb_ref,
                src_ref, sslot_ref, gn_ref):
    x = x_ref[...]
    logits = jnp.dot(x, rw_ref[...], preferred_element_type=jnp.float32)
    logits = logits + rb_ref[...]
    m = jnp.max(logits, axis=1, keepdims=True)
    ex = jnp.exp(logits - m)
    gates = ex / jnp.sum(ex, axis=1, keepdims=True)

    lane = lax.broadcasted_iota(jnp.int32, (N, E), 1)
    g1 = jnp.max(gates, axis=1, keepdims=True)
    i1 = jnp.min(jnp.where(gates == g1, lane, E), axis=1, keepdims=True)
    masked = jnp.where(lane == i1, -1.0, gates)
    g2 = jnp.max(masked, axis=1, keepdims=True)
    i2 = jnp.min(jnp.where(masked == g2, lane, E), axis=1, keepdims=True)

    oh1 = (lane == i1).astype(jnp.float32)
    oh2 = (lane == i2).astype(jnp.float32)
    cnt = oh1 + oh2

    srow = lax.broadcasted_iota(jnp.int32, (TBLK, TBLK), 0)
    scol = lax.broadcasted_iota(jnp.int32, (TBLK, TBLK), 1)
    tril = (scol < srow).astype(jnp.float32)
    brow = lax.broadcasted_iota(jnp.int32, (NTB, NTB), 0)
    bcol = lax.broadcasted_iota(jnp.int32, (NTB, NTB), 1)
    btril = (bcol < brow).astype(jnp.float32)

    ones_row = jnp.ones((1, TBLK), jnp.float32)
    bsums = []
    cums = []
    for b in range(NTB):
        blk = lax.slice(cnt, (b * TBLK, 0), ((b + 1) * TBLK, E))
        cums.append(jnp.dot(tril, blk, preferred_element_type=jnp.float32))
        bsums.append(jnp.dot(ones_row, blk, preferred_element_type=jnp.float32))
    bs = jnp.concatenate(bsums, axis=0)
    off = jnp.dot(btril, bs, preferred_element_type=jnp.float32)
    cume = jnp.concatenate(
        [cums[b] + lax.slice(off, (b, 0), (b + 1, E)) for b in range(NTB)],
        axis=0)

    p1 = jnp.sum(oh1 * cume, axis=1, keepdims=True)
    p2 = jnp.sum(oh2 * (cume + oh1), axis=1, keepdims=True)

    capf = jnp.float32(CAP)
    keep1 = p1 < capf
    keep2 = p2 < capf
    slot1 = jnp.where(keep1, i1.astype(jnp.float32) * capf + p1,
                      jnp.float32(ECAP)).astype(jnp.int32)
    slot2 = jnp.where(keep2, i2.astype(jnp.float32) * capf + p2,
                      jnp.float32(ECAP)).astype(jnp.int32)
    sslot1 = (i1.astype(jnp.float32) * capf
              + (p1 - jnp.floor(p1 / capf) * capf)).astype(jnp.int32)
    sslot2 = (i2.astype(jnp.float32) * capf
              + (p2 - jnp.floor(p2 / capf) * capf)).astype(jnp.int32)
    ge1 = jnp.where(keep1, g1, 0.0)
    ge2 = jnp.where(keep2, g2, 0.0)
    wsum = ge1 + ge2
    denom = jnp.maximum(wsum, 1e-6)
    gn1 = jnp.where(wsum > 0, ge1 / denom, 0.0)
    gn2 = jnp.where(wsum > 0, ge2 / denom, 0.0)

    sslot_ref[0:N] = sslot1
    sslot_ref[N:2 * N] = sslot2
    gn_ref[0:N] = gn1
    gn_ref[N:2 * N] = gn2

    tok = lax.broadcasted_iota(jnp.int32, (N, SBLK), 0)
    for c in range(NSB):
        sid = c * SBLK + lax.broadcasted_iota(jnp.int32, (N, SBLK), 1)
        eq1 = slot1 == sid
        eq2 = slot2 == sid
        ssum = (jnp.sum(jnp.where(eq1, tok, 0), axis=0, keepdims=True)
                + jnp.sum(jnp.where(eq2, tok, 0), axis=0, keepdims=True))
        occ = (jnp.sum(eq1.astype(jnp.int32), axis=0, keepdims=True)
               + jnp.sum(eq2.astype(jnp.int32), axis=0, keepdims=True))
        src_ref[0:1, pl.ds(c * SBLK, SBLK)] = jnp.where(occ > 0, ssum, 0)


def _route(xf, router_w, router_b):
    return pl.pallas_call(
        _route_body,
        out_shape=(
            jax.ShapeDtypeStruct((1, ECAP), jnp.int32),
            jax.ShapeDtypeStruct((2 * N, 1), jnp.int32),
            jax.ShapeDtypeStruct((2 * N, 1), jnp.float32),
        ),
    )(xf, router_w, router_b.reshape(1, E))


def _make_gather(n_rows, chunk, clamp, width=D):
    info = plsc.get_sparse_core_info()
    nc, ns = info.num_cores, info.num_subcores
    nw = nc * ns
    rpw = n_rows // nw
    nch = rpw // chunk
    mesh = plsc.VectorSubcoreMesh(core_axis_name="c", subcore_axis_name="s")

    @functools.partial(
        pl.kernel,
        out_type=jax.ShapeDtypeStruct((n_rows, width), jnp.float32),
        mesh=mesh,
        scratch_types=[
            pltpu.VMEM((chunk,), jnp.int32),
            pltpu.VMEM((chunk, width), jnp.float32),
            pltpu.SemaphoreType.DMA,
        ],
    )
    def gather(table_hbm, idx_hbm, out_hbm, idx_v, rows_v, sem):
        wid = lax.axis_index("s") * nc + lax.axis_index("c")

        for k in range(nch):
            base = wid * rpw + k * chunk
            pltpu.sync_copy(idx_hbm.at[pl.ds(base, chunk)], idx_v)
            if clamp is not None:
                for j in range(chunk // 16):
                    sl = pl.ds(j * 16, 16)
                    idx_v[sl] = jnp.minimum(idx_v[sl], clamp)
            pltpu.async_copy(table_hbm.at[idx_v], rows_v, sem).wait()
            pltpu.sync_copy(rows_v, out_hbm.at[pl.ds(base, chunk)])

    return gather


_SQRT_HALF = 1.0 / math.sqrt(2.0)


def _ffn_body(buf_ref, w1_ref, b1_ref, w2_ref, b2_ref, out_ref, acc_ref):
    f = pl.program_id(1)
    h = jnp.dot(buf_ref[...], w1_ref[0], preferred_element_type=jnp.float32)
    h = h + b1_ref[0]
    h = 0.5 * h * (1.0 + lax.erf(h * _SQRT_HALF))
    acc = jnp.dot(h, w2_ref[0], preferred_element_type=jnp.float32)

    @pl.when(f == 0)
    def _():
        acc_ref[...] = acc + b2_ref[0]

    @pl.when(f != 0)
    def _():
        acc_ref[...] = acc_ref[...] + acc

    @pl.when(f == NFB - 1)
    def _():
        out_ref[...] = acc_ref[...]


def _ffn(buf, w1, b1, w2, b2):
    return pl.pallas_call(
        _ffn_body,
        grid=(E, NFB),
        in_specs=[
            pl.BlockSpec((CAP, D), lambda e, f: (e, 0)),
            pl.BlockSpec((1, D, FBLK), lambda e, f: (e, 0, f)),
            pl.BlockSpec((1, 1, FBLK), lambda e, f: (e, 0, f)),
            pl.BlockSpec((1, FBLK, D), lambda e, f: (e, f, 0)),
            pl.BlockSpec((1, 1, D), lambda e, f: (e, 0, 0)),
        ],
        out_specs=pl.BlockSpec((CAP, D), lambda e, f: (e, 0)),
        out_shape=jax.ShapeDtypeStruct((ECAP, D), jnp.float32),
        scratch_shapes=[pltpu.VMEM((CAP, D), jnp.float32)],
    )(buf, w1, b1.reshape(E, 1, F), w2, b2.reshape(E, 1, D))


def _make_combine():
    info = plsc.get_sparse_core_info()
    nc, ns = info.num_cores, info.num_subcores
    nw = nc * ns
    tpw = N // nw
    ch = 16
    nch = tpw // ch
    mesh = plsc.VectorSubcoreMesh(core_axis_name="c", subcore_axis_name="s")

    @functools.partial(
        pl.kernel,
        out_type=jax.ShapeDtypeStruct((N, D), jnp.float32),
        mesh=mesh,
        compiler_params=pltpu.CompilerParams(needs_layout_passes=False),
        scratch_types=[
            pltpu.VMEM((tpw,), jnp.int32),
            pltpu.VMEM((tpw,), jnp.int32),
            pltpu.VMEM((tpw,), jnp.float32),
            pltpu.VMEM((tpw,), jnp.float32),
            pltpu.VMEM((ch, D), jnp.float32),
            pltpu.VMEM((ch, D), jnp.float32),
            pltpu.VMEM((ch, D), jnp.float32),
            pltpu.VMEM((ch, D), jnp.float32),
            pltpu.SemaphoreType.DMA,
            pltpu.SemaphoreType.DMA,
            pltpu.SemaphoreType.DMA,
            pltpu.SemaphoreType.DMA,
        ],
    )
    def combine(out_hbm, sslot_hbm, gn_hbm, y_hbm,
                i1_v, i2_v, g1_v, g2_v, r1a, r1b, r2a, r2b,
                s1a, s1b, s2a, s2b):
        wid = lax.axis_index("s") * nc + lax.axis_index("c")
        base = wid * tpw
        pltpu.sync_copy(sslot_hbm.at[pl.ds(base, tpw)], i1_v)
        pltpu.sync_copy(sslot_hbm.at[pl.ds(N + base, tpw)], i2_v)
        pltpu.sync_copy(gn_hbm.at[pl.ds(base, tpw)], g1_v)
        pltpu.sync_copy(gn_hbm.at[pl.ds(N + base, tpw)], g2_v)

        r1 = [r1a, r1b]
        r2 = [r2a, r2b]
        s1 = [s1a, s1b]
        s2 = [s2a, s2b]
        rowi = lax.iota(jnp.int32, 16)

        def issue(k):
            sl = k % 2
            return (pltpu.async_copy(out_hbm.at[i1_v.at[pl.ds(k * ch, ch)]],
                                     r1[sl], s1[sl]),
                    pltpu.async_copy(out_hbm.at[i2_v.at[pl.ds(k * ch, ch)]],
                                     r2[sl], s2[sl]))

        pend = issue(0)
        for k in range(nch):
            sl = k % 2
            nxt = issue(k + 1) if k + 1 < nch else None
            pend[0].wait()
            pend[1].wait()
            ga = g1_v[pl.ds(k * ch, 16)]
            gb = g2_v[pl.ds(k * ch, 16)]

            def body(c8, _):
                for j in range(8):
                    col = jnp.full((16,), c8 * 8 + j, jnp.int32)
                    a = plsc.load_gather(r1[sl], [rowi, col])
                    b = plsc.load_gather(r2[sl], [rowi, col])
                    plsc.store_scatter(r1[sl], [rowi, col], ga * a + gb * b)
                return 0

            lax.fori_loop(0, D // 8, body, 0)
            pltpu.sync_copy(r1[sl], y_hbm.at[pl.ds(base + k * ch, ch)])
            pend = nxt

    return combine


def kernel(x, router_w, router_b, w1, b1, w2, b2):
    xf = x.reshape(N, D).astype(jnp.float32)
    src, sslot, gncat = _route(xf, router_w, router_b)

    buf = _make_gather(ECAP, 80, None)(xf, src.reshape(ECAP))
    out_flat = _ffn(buf, w1, b1, w2, b2)
    y = _make_combine()(out_flat, sslot.reshape(2 * N), gncat.reshape(2 * N))
    return y.reshape(x.shape)


# speedup vs baseline: 1.7197x; 1.7197x over previous
import functools
import math

import jax
import jax.numpy as jnp
from jax import lax
from jax.experimental import pallas as pl
from jax.experimental.pallas import tpu as pltpu
from jax.experimental.pallas import tpu_sc as plsc

N = 2048
D = 1024
F = 4096
E = 8
CAP = 320
ECAP = E * CAP
FBLK = 2048
NFB = F // FBLK
TBLK = 128
NTB = N // TBLK
SBLK = 256
NSB = ECAP // SBLK


def _route_body(x_ref, rw_ref, rb_ref,
                src_ref, sslot_ref, gn_ref):
    x = x_ref[...]
    logits = jnp.dot(x, rw_ref[...], preferred_element_type=jnp.float32)
    logits = logits + rb_ref[...]
    m = jnp.max(logits, axis=1, keepdims=True)
    ex = jnp.exp(logits - m)
    gates = ex / jnp.sum(ex, axis=1, keepdims=True)

    lane = lax.broadcasted_iota(jnp.int32, (N, E), 1)
    g1 = jnp.max(gates, axis=1, keepdims=True)
    i1 = jnp.min(jnp.where(gates == g1, lane, E), axis=1, keepdims=True)
    masked = jnp.where(lane == i1, -1.0, gates)
    g2 = jnp.max(masked, axis=1, keepdims=True)
    i2 = jnp.min(jnp.where(masked == g2, lane, E), axis=1, keepdims=True)

    oh1 = (lane == i1).astype(jnp.float32)
    oh2 = (lane == i2).astype(jnp.float32)
    cnt = oh1 + oh2

    srow = lax.broadcasted_iota(jnp.int32, (TBLK, TBLK), 0)
    scol = lax.broadcasted_iota(jnp.int32, (TBLK, TBLK), 1)
    tril = (scol < srow).astype(jnp.float32)
    brow = lax.broadcasted_iota(jnp.int32, (NTB, NTB), 0)
    bcol = lax.broadcasted_iota(jnp.int32, (NTB, NTB), 1)
    btril = (bcol < brow).astype(jnp.float32)

    ones_row = jnp.ones((1, TBLK), jnp.float32)
    bsums = []
    cums = []
    for b in range(NTB):
        blk = lax.slice(cnt, (b * TBLK, 0), ((b + 1) * TBLK, E))
        cums.append(jnp.dot(tril, blk, preferred_element_type=jnp.float32))
        bsums.append(jnp.dot(ones_row, blk, preferred_element_type=jnp.float32))
    bs = jnp.concatenate(bsums, axis=0)
    off = jnp.dot(btril, bs, preferred_element_type=jnp.float32)
    cume = jnp.concatenate(
        [cums[b] + lax.slice(off, (b, 0), (b + 1, E)) for b in range(NTB)],
        axis=0)

    p1 = jnp.sum(oh1 * cume, axis=1, keepdims=True)
    p2 = jnp.sum(oh2 * (cume + oh1), axis=1, keepdims=True)

    capf = jnp.float32(CAP)
    keep1 = p1 < capf
    keep2 = p2 < capf
    slot1 = jnp.where(keep1, i1.astype(jnp.float32) * capf + p1,
                      jnp.float32(ECAP)).astype(jnp.int32)
    slot2 = jnp.where(keep2, i2.astype(jnp.float32) * capf + p2,
                      jnp.float32(ECAP)).astype(jnp.int32)
    sslot1 = (i1.astype(jnp.float32) * capf
              + (p1 - jnp.floor(p1 / capf) * capf)).astype(jnp.int32)
    sslot2 = (i2.astype(jnp.float32) * capf
              + (p2 - jnp.floor(p2 / capf) * capf)).astype(jnp.int32)
    ge1 = jnp.where(keep1, g1, 0.0)
    ge2 = jnp.where(keep2, g2, 0.0)
    wsum = ge1 + ge2
    denom = jnp.maximum(wsum, 1e-6)
    gn1 = jnp.where(wsum > 0, ge1 / denom, 0.0)
    gn2 = jnp.where(wsum > 0, ge2 / denom, 0.0)

    sslot_ref[0:N] = sslot1
    sslot_ref[N:2 * N] = sslot2
    gn_ref[0:N] = gn1
    gn_ref[N:2 * N] = gn2

    tok = lax.broadcasted_iota(jnp.int32, (N, SBLK), 0)
    for c in range(NSB):
        sid = c * SBLK + lax.broadcasted_iota(jnp.int32, (N, SBLK), 1)
        eq1 = slot1 == sid
        eq2 = slot2 == sid
        ssum = (jnp.sum(jnp.where(eq1, tok, 0), axis=0, keepdims=True)
                + jnp.sum(jnp.where(eq2, tok, 0), axis=0, keepdims=True))
        occ = (jnp.sum(eq1.astype(jnp.int32), axis=0, keepdims=True)
               + jnp.sum(eq2.astype(jnp.int32), axis=0, keepdims=True))
        src_ref[0:1, pl.ds(c * SBLK, SBLK)] = jnp.where(occ > 0, ssum, 0)


def _route(xf, router_w, router_b):
    return pl.pallas_call(
        _route_body,
        out_shape=(
            jax.ShapeDtypeStruct((1, ECAP), jnp.int32),
            jax.ShapeDtypeStruct((2 * N, 1), jnp.int32),
            jax.ShapeDtypeStruct((2 * N, 1), jnp.float32),
        ),
    )(xf, router_w, router_b.reshape(1, E))


def _make_gather(n_rows, chunk, clamp, width=D):
    info = plsc.get_sparse_core_info()
    nc, ns = info.num_cores, info.num_subcores
    nw = nc * ns
    rpw = n_rows // nw
    nch = rpw // chunk
    mesh = plsc.VectorSubcoreMesh(core_axis_name="c", subcore_axis_name="s")

    @functools.partial(
        pl.kernel,
        out_type=jax.ShapeDtypeStruct((n_rows, width), jnp.float32),
        mesh=mesh,
        scratch_types=[
            pltpu.VMEM((chunk,), jnp.int32),
            pltpu.VMEM((chunk, width), jnp.float32),
            pltpu.SemaphoreType.DMA,
        ],
    )
    def gather(table_hbm, idx_hbm, out_hbm, idx_v, rows_v, sem):
        wid = lax.axis_index("s") * nc + lax.axis_index("c")

        for k in range(nch):
            base = wid * rpw + k * chunk
            pltpu.sync_copy(idx_hbm.at[pl.ds(base, chunk)], idx_v)
            if clamp is not None:
                for j in range(chunk // 16):
                    sl = pl.ds(j * 16, 16)
                    idx_v[sl] = jnp.minimum(idx_v[sl], clamp)
            pltpu.async_copy(table_hbm.at[idx_v], rows_v, sem).wait()
            pltpu.sync_copy(rows_v, out_hbm.at[pl.ds(base, chunk)])

    return gather


_SQRT_HALF = 1.0 / math.sqrt(2.0)


def _ffn_body(buf_ref, w1_ref, b1_ref, w2_ref, b2_ref, out_ref, acc_ref):
    f = pl.program_id(1)
    h = jnp.dot(buf_ref[...], w1_ref[0], preferred_element_type=jnp.float32)
    h = h + b1_ref[0]
    h = 0.5 * h * (1.0 + lax.erf(h * _SQRT_HALF))
    acc = jnp.dot(h, w2_ref[0], preferred_element_type=jnp.float32)

    @pl.when(f == 0)
    def _():
        acc_ref[...] = acc + b2_ref[0]

    @pl.when(f != 0)
    def _():
        acc_ref[...] = acc_ref[...] + acc

    @pl.when(f == NFB - 1)
    def _():
        a = acc_ref[...]
        lo = lax.bitcast_convert_type(
            lax.slice(a, (0, 0), (CAP, D // 2)), jnp.uint32)
        hi = lax.bitcast_convert_type(
            lax.slice(a, (0, D // 2), (CAP, D)), jnp.uint32)

        def rnd(u):
            return u + jnp.uint32(0x7FFF) + ((u >> jnp.uint32(16))
                                             & jnp.uint32(1))

        w = ((rnd(lo) >> jnp.uint32(16))
             | (rnd(hi) & jnp.uint32(0xFFFF0000)))
        out_ref[...] = lax.bitcast_convert_type(w, jnp.float32)


def _ffn(buf, w1, b1, w2, b2):
    return pl.pallas_call(
        _ffn_body,
        grid=(E, NFB),
        in_specs=[
            pl.BlockSpec((CAP, D), lambda e, f: (e, 0)),
            pl.BlockSpec((1, D, FBLK), lambda e, f: (e, 0, f)),
            pl.BlockSpec((1, 1, FBLK), lambda e, f: (e, 0, f)),
            pl.BlockSpec((1, FBLK, D), lambda e, f: (e, f, 0)),
            pl.BlockSpec((1, 1, D), lambda e, f: (e, 0, 0)),
        ],
        out_specs=pl.BlockSpec((CAP, D // 2), lambda e, f: (e, 0)),
        out_shape=jax.ShapeDtypeStruct((ECAP, D // 2), jnp.float32),
        scratch_shapes=[pltpu.VMEM((CAP, D), jnp.float32)],
    )(buf, w1, b1.reshape(E, 1, F), w2, b2.reshape(E, 1, D))


CBLK = 256
NCB = N // CBLK


def _combine_body(r1_ref, r2_ref, g1_ref, g2_ref, y_ref):
    w1 = lax.bitcast_convert_type(r1_ref[...], jnp.uint32)
    w2 = lax.bitcast_convert_type(r2_ref[...], jnp.uint32)
    g1 = g1_ref[...]
    g2 = g2_ref[...]

    def unlo(w):
        return lax.bitcast_convert_type(w << jnp.uint32(16), jnp.float32)

    def unhi(w):
        return lax.bitcast_convert_type(w & jnp.uint32(0xFFFF0000),
                                        jnp.float32)

    y_ref[:, 0:D // 2] = g1 * unlo(w1) + g2 * unlo(w2)
    y_ref[:, D // 2:D] = g1 * unhi(w1) + g2 * unhi(w2)


def _combine(rcat, gncat):
    return pl.pallas_call(
        _combine_body,
        grid=(NCB,),
        in_specs=[
            pl.BlockSpec((CBLK, D // 2), lambda b: (b, 0)),
            pl.BlockSpec((CBLK, D // 2), lambda b: (b + NCB, 0)),
            pl.BlockSpec((CBLK, 1), lambda b: (b, 0)),
            pl.BlockSpec((CBLK, 1), lambda b: (b + NCB, 0)),
        ],
        out_specs=pl.BlockSpec((CBLK, D), lambda b: (b, 0)),
        out_shape=jax.ShapeDtypeStruct((N, D), jnp.float32),
    )(rcat, rcat, gncat, gncat)


def kernel(x, router_w, router_b, w1, b1, w2, b2):
    xf = x.reshape(N, D).astype(jnp.float32)
    src, sslot, gncat = _route(xf, router_w, router_b)

    buf = _make_gather(ECAP, 80, None)(xf, src.reshape(ECAP))
    out_p = _ffn(buf, w1, b1, w2, b2)
    rcat = _make_gather(2 * N, 128, None, D // 2)(out_p, sslot.reshape(2 * N))
    y = _combine(rcat, gncat)
    return y.reshape(x.shape)


# speedup vs baseline: 1.7897x; 1.0407x over previous
import functools
import math

import jax
import jax.numpy as jnp
from jax import lax
from jax.experimental import pallas as pl
from jax.experimental.pallas import tpu as pltpu
from jax.experimental.pallas import tpu_sc as plsc

N = 2048
D = 1024
F = 4096
E = 8
CAP = 320
ECAP = E * CAP
FBLK = 2048
NFB = F // FBLK
TBLK = 128
NTB = N // TBLK
SBLK = 256
NSB = ECAP // SBLK


def _pack_bf16(a, rows):
    lo = lax.bitcast_convert_type(
        lax.slice(a, (0, 0), (rows, D // 2)), jnp.uint32)
    hi = lax.bitcast_convert_type(
        lax.slice(a, (0, D // 2), (rows, D)), jnp.uint32)

    def rnd(u):
        return u + jnp.uint32(0x7FFF) + ((u >> jnp.uint32(16)) & jnp.uint32(1))

    w = (rnd(lo) >> jnp.uint32(16)) | (rnd(hi) & jnp.uint32(0xFFFF0000))
    return lax.bitcast_convert_type(w, jnp.float32)


def _unpack_bf16(p):
    w = lax.bitcast_convert_type(p, jnp.uint32)
    lo = lax.bitcast_convert_type(w << jnp.uint32(16), jnp.float32)
    hi = lax.bitcast_convert_type(w & jnp.uint32(0xFFFF0000), jnp.float32)
    return jnp.concatenate([lo, hi], axis=1)


def _route_body(x_ref, rw_ref, rb_ref,
                src_ref, xp_ref, sslot_ref, gn_ref):
    x = x_ref[...]
    xp_ref[...] = _pack_bf16(x, N)
    logits = jnp.dot(x, rw_ref[...], preferred_element_type=jnp.float32)
    logits = logits + rb_ref[...]
    m = jnp.max(logits, axis=1, keepdims=True)
    ex = jnp.exp(logits - m)
    gates = ex / jnp.sum(ex, axis=1, keepdims=True)

    lane = lax.broadcasted_iota(jnp.int32, (N, E), 1)
    g1 = jnp.max(gates, axis=1, keepdims=True)
    i1 = jnp.min(jnp.where(gates == g1, lane, E), axis=1, keepdims=True)
    masked = jnp.where(lane == i1, -1.0, gates)
    g2 = jnp.max(masked, axis=1, keepdims=True)
    i2 = jnp.min(jnp.where(masked == g2, lane, E), axis=1, keepdims=True)

    oh1 = (lane == i1).astype(jnp.float32)
    oh2 = (lane == i2).astype(jnp.float32)
    cnt = oh1 + oh2

    srow = lax.broadcasted_iota(jnp.int32, (TBLK, TBLK), 0)
    scol = lax.broadcasted_iota(jnp.int32, (TBLK, TBLK), 1)
    tril = (scol < srow).astype(jnp.float32)
    brow = lax.broadcasted_iota(jnp.int32, (NTB, NTB), 0)
    bcol = lax.broadcasted_iota(jnp.int32, (NTB, NTB), 1)
    btril = (bcol < brow).astype(jnp.float32)

    ones_row = jnp.ones((1, TBLK), jnp.float32)
    bsums = []
    cums = []
    for b in range(NTB):
        blk = lax.slice(cnt, (b * TBLK, 0), ((b + 1) * TBLK, E))
        cums.append(jnp.dot(tril, blk, preferred_element_type=jnp.float32))
        bsums.append(jnp.dot(ones_row, blk, preferred_element_type=jnp.float32))
    bs = jnp.concatenate(bsums, axis=0)
    off = jnp.dot(btril, bs, preferred_element_type=jnp.float32)
    cume = jnp.concatenate(
        [cums[b] + lax.slice(off, (b, 0), (b + 1, E)) for b in range(NTB)],
        axis=0)

    p1 = jnp.sum(oh1 * cume, axis=1, keepdims=True)
    p2 = jnp.sum(oh2 * (cume + oh1), axis=1, keepdims=True)

    capf = jnp.float32(CAP)
    keep1 = p1 < capf
    keep2 = p2 < capf
    slot1 = jnp.where(keep1, i1.astype(jnp.float32) * capf + p1,
                      jnp.float32(ECAP)).astype(jnp.int32)
    slot2 = jnp.where(keep2, i2.astype(jnp.float32) * capf + p2,
                      jnp.float32(ECAP)).astype(jnp.int32)
    sslot1 = (i1.astype(jnp.float32) * capf
              + (p1 - jnp.floor(p1 / capf) * capf)).astype(jnp.int32)
    sslot2 = (i2.astype(jnp.float32) * capf
              + (p2 - jnp.floor(p2 / capf) * capf)).astype(jnp.int32)
    ge1 = jnp.where(keep1, g1, 0.0)
    ge2 = jnp.where(keep2, g2, 0.0)
    wsum = ge1 + ge2
    denom = jnp.maximum(wsum, 1e-6)
    gn1 = jnp.where(wsum > 0, ge1 / denom, 0.0)
    gn2 = jnp.where(wsum > 0, ge2 / denom, 0.0)

    sslot_ref[0:N] = sslot1
    sslot_ref[N:2 * N] = sslot2
    gn_ref[0:N] = gn1
    gn_ref[N:2 * N] = gn2

    tok = lax.broadcasted_iota(jnp.int32, (N, SBLK), 0)
    for c in range(NSB):
        sid = c * SBLK + lax.broadcasted_iota(jnp.int32, (N, SBLK), 1)
        eq1 = slot1 == sid
        eq2 = slot2 == sid
        ssum = (jnp.sum(jnp.where(eq1, tok, 0), axis=0, keepdims=True)
                + jnp.sum(jnp.where(eq2, tok, 0), axis=0, keepdims=True))
        occ = (jnp.sum(eq1.astype(jnp.int32), axis=0, keepdims=True)
               + jnp.sum(eq2.astype(jnp.int32), axis=0, keepdims=True))
        src_ref[0:1, pl.ds(c * SBLK, SBLK)] = jnp.where(occ > 0, ssum, 0)


def _route(xf, router_w, router_b):
    return pl.pallas_call(
        _route_body,
        out_shape=(
            jax.ShapeDtypeStruct((1, ECAP), jnp.int32),
            jax.ShapeDtypeStruct((N, D // 2), jnp.float32),
            jax.ShapeDtypeStruct((2 * N, 1), jnp.int32),
            jax.ShapeDtypeStruct((2 * N, 1), jnp.float32),
        ),
    )(xf, router_w, router_b.reshape(1, E))


def _make_gather(n_rows, chunk, clamp, width=D):
    info = plsc.get_sparse_core_info()
    nc, ns = info.num_cores, info.num_subcores
    nw = nc * ns
    rpw = n_rows // nw
    nch = rpw // chunk
    mesh = plsc.VectorSubcoreMesh(core_axis_name="c", subcore_axis_name="s")

    @functools.partial(
        pl.kernel,
        out_type=jax.ShapeDtypeStruct((n_rows, width), jnp.float32),
        mesh=mesh,
        scratch_types=[
            pltpu.VMEM((chunk,), jnp.int32),
            pltpu.VMEM((chunk, width), jnp.float32),
            pltpu.SemaphoreType.DMA,
        ],
    )
    def gather(table_hbm, idx_hbm, out_hbm, idx_v, rows_v, sem):
        wid = lax.axis_index("s") * nc + lax.axis_index("c")

        for k in range(nch):
            base = wid * rpw + k * chunk
            pltpu.sync_copy(idx_hbm.at[pl.ds(base, chunk)], idx_v)
            if clamp is not None:
                for j in range(chunk // 16):
                    sl = pl.ds(j * 16, 16)
                    idx_v[sl] = jnp.minimum(idx_v[sl], clamp)
            pltpu.async_copy(table_hbm.at[idx_v], rows_v, sem).wait()
            pltpu.sync_copy(rows_v, out_hbm.at[pl.ds(base, chunk)])

    return gather


_SQRT_HALF = 1.0 / math.sqrt(2.0)


def _ffn_body(buf_ref, w1_ref, b1_ref, w2_ref, b2_ref, out_ref, acc_ref):
    f = pl.program_id(1)
    xb = _unpack_bf16(buf_ref[...])
    h = jnp.dot(xb, w1_ref[0], preferred_element_type=jnp.float32)
    h = h + b1_ref[0]
    h = 0.5 * h * (1.0 + lax.erf(h * _SQRT_HALF))
    acc = jnp.dot(h, w2_ref[0], preferred_element_type=jnp.float32)

    @pl.when(f == 0)
    def _():
        acc_ref[...] = acc + b2_ref[0]

    @pl.when(f != 0)
    def _():
        acc_ref[...] = acc_ref[...] + acc

    @pl.when(f == NFB - 1)
    def _():
        out_ref[...] = _pack_bf16(acc_ref[...], CAP)


def _ffn(buf, w1, b1, w2, b2):
    return pl.pallas_call(
        _ffn_body,
        grid=(E, NFB),
        in_specs=[
            pl.BlockSpec((CAP, D // 2), lambda e, f: (e, 0)),
            pl.BlockSpec((1, D, FBLK), lambda e, f: (e, 0, f)),
            pl.BlockSpec((1, 1, FBLK), lambda e, f: (e, 0, f)),
            pl.BlockSpec((1, FBLK, D), lambda e, f: (e, f, 0)),
            pl.BlockSpec((1, 1, D), lambda e, f: (e, 0, 0)),
        ],
        out_specs=pl.BlockSpec((CAP, D // 2), lambda e, f: (e, 0)),
        out_shape=jax.ShapeDtypeStruct((ECAP, D // 2), jnp.float32),
        scratch_shapes=[pltpu.VMEM((CAP, D), jnp.float32)],
    )(buf, w1, b1.reshape(E, 1, F), w2, b2.reshape(E, 1, D))


CBLK = 256
NCB = N // CBLK


def _combine_body(r1_ref, r2_ref, g1_ref, g2_ref, y_ref):
    w1 = lax.bitcast_convert_type(r1_ref[...], jnp.uint32)
    w2 = lax.bitcast_convert_type(r2_ref[...], jnp.uint32)
    g1 = g1_ref[...]
    g2 = g2_ref[...]

    def unlo(w):
        return lax.bitcast_convert_type(w << jnp.uint32(16), jnp.float32)

    def unhi(w):
        return lax.bitcast_convert_type(w & jnp.uint32(0xFFFF0000),
                                        jnp.float32)

    y_ref[:, 0:D // 2] = g1 * unlo(w1) + g2 * unlo(w2)
    y_ref[:, D // 2:D] = g1 * unhi(w1) + g2 * unhi(w2)


def _combine(rcat, gncat):
    return pl.pallas_call(
        _combine_body,
        grid=(NCB,),
        in_specs=[
            pl.BlockSpec((CBLK, D // 2), lambda b: (b, 0)),
            pl.BlockSpec((CBLK, D // 2), lambda b: (b + NCB, 0)),
            pl.BlockSpec((CBLK, 1), lambda b: (b, 0)),
            pl.BlockSpec((CBLK, 1), lambda b: (b + NCB, 0)),
        ],
        out_specs=pl.BlockSpec((CBLK, D), lambda b: (b, 0)),
        out_shape=jax.ShapeDtypeStruct((N, D), jnp.float32),
    )(rcat, rcat, gncat, gncat)


def kernel(x, router_w, router_b, w1, b1, w2, b2):
    xf = x.reshape(N, D).astype(jnp.float32)
    src, xp, sslot, gncat = _route(xf, router_w, router_b)

    buf = _make_gather(ECAP, 80, None, D // 2)(xp, src.reshape(ECAP))
    out_p = _ffn(buf, w1, b1, w2, b2)
    rcat = _make_gather(2 * N, 128, None, D // 2)(out_p, sslot.reshape(2 * N))
    y = _combine(rcat, gncat)
    return y.reshape(x.shape)
